# baseline (speedup 1.0000x reference)
import functools

import jax
import jax.numpy as jnp
from jax.experimental import pallas as pl
from jax.experimental.pallas import tpu as pltpu

_GRID_RANGE = ((0.0, 1.0), (0.0, 1.0))
_GRID_SHAPE = (64, 64)


def _setconv_kernel(xt_ref, zt_ref, ls_ref, xg_ref, out_ref):
    dxp, mn = xt_ref.shape
    mm, gi, gjdz = out_ref.shape
    n = mn // mm
    dz = zt_ref.shape[0]
    gj = gjdz // dz
    step0 = 1.0 / (gi - 1)
    step1 = 1.0 / (gj - 1)

    p = ls_ref[0, :]
    ls = 1e-5 + jnp.logaddexp(p, 0.0)
    inv = 1.0 / (ls * ls)
    inv0 = inv[0]
    inv1 = inv[1]

    x0_row = xt_ref[0:1, :]
    x1_col = jnp.transpose(xt_ref[1:2, :], (1, 0))

    jlane = jax.lax.broadcasted_iota(jnp.int32, (1, gi), 1)
    ax1_row = jlane.astype(jnp.float32) * step1
    ax0_col = jax.lax.broadcasted_iota(
        jnp.int32, (gi, 1), 0).astype(jnp.float32) * step0

    d0 = ax0_col - x0_row
    a = jnp.exp(-0.5 * inv0 * d0 * d0)

    d1 = x1_col - ax1_row
    bs = jnp.exp(-0.5 * inv1 * d1 * d1)

    q2 = jax.lax.broadcasted_iota(jnp.int32, (gj, gjdz), 1)
    r2 = jax.lax.broadcasted_iota(jnp.int32, (gj, gjdz), 0)
    e2 = jnp.where((q2 // dz) == r2, 1.0, 0.0)
    q3 = jax.lax.broadcasted_iota(jnp.int32, (dz, 128), 1)
    r3 = jax.lax.broadcasted_iota(jnp.int32, (dz, 128), 0)
    e3 = jnp.where((q3 % dz) == r3, 1.0, 0.0)

    b_rep = jnp.dot(bs, e2, preferred_element_type=jnp.float32,
                    precision=jax.lax.Precision.DEFAULT)
    z128 = jax.lax.dot_general(
        zt_ref[...], e3, (((0,), (0,)), ((), ())),
        preferred_element_type=jnp.float32,
        precision=jax.lax.Precision.DEFAULT)
    z_tile = jnp.concatenate([z128] * (gjdz // 128), axis=1)

    t = b_rep * z_tile

    qg = jax.lax.broadcasted_iota(jnp.int32, (gi, gj * dxp), 1)
    rg = jax.lax.broadcasted_iota(jnp.int32, (gi, gj * dxp), 0)
    gx = jnp.where(qg % dxp == 0, rg.astype(jnp.float32) * step0,
                   (qg // dxp).astype(jnp.float32) * step1)

    for b in range(mm):
        out_ref[b] = jax.lax.dot_general(
            a[:, b * n:(b + 1) * n], t[b * n:(b + 1) * n],
            (((1,), (0,)), ((), ())),
            preferred_element_type=jnp.float32,
            precision=jax.lax.Precision.DEFAULT)
        xg_ref[b] = gx


@functools.partial(jax.jit, static_argnames=())
def kernel(x, z, lengthscale_param):
    m, n, dx = x.shape
    dz = z.shape[-1]
    gi, gj = _GRID_SHAPE

    xt = x.reshape(m * n, dx).T
    zt = z.reshape(m * n, dz).T
    ls2 = lengthscale_param.reshape(1, dx)

    mh = m // 2
    xg, out = pl.pallas_call(
        _setconv_kernel,
        grid=(2,),
        in_specs=[
            pl.BlockSpec((dx, mh * n), lambda g: (0, g)),
            pl.BlockSpec((dz, mh * n), lambda g: (0, g)),
            pl.BlockSpec((1, dx), lambda g: (0, 0)),
        ],
        out_specs=[
            pl.BlockSpec((mh, gi, gj * dx), lambda g: (g, 0, 0)),
            pl.BlockSpec((mh, gi, gj * dz), lambda g: (g, 0, 0)),
        ],
        out_shape=[
            jax.ShapeDtypeStruct((m, gi, gj * dx), jnp.float32),
            jax.ShapeDtypeStruct((m, gi, gj * dz), jnp.float32),
        ],
        compiler_params=pltpu.CompilerParams(
            dimension_semantics=("parallel",),
        ),
    )(xt, zt, ls2)

    x_grid = xg.reshape(m, gi, gj, dx)
    z_grid = out.reshape(m, gi, gj, dz)
    return (x_grid, z_grid)


# speedup vs baseline: 1.0186x; 1.0186x over previous
import functools

import jax
import jax.numpy as jnp
from jax.experimental import pallas as pl
from jax.experimental.pallas import tpu as pltpu

_GRID_RANGE = ((0.0, 1.0), (0.0, 1.0))
_GRID_SHAPE = (64, 64)


def _setconv_kernel(xt_ref, zt_ref, ls_ref, xg_ref, out_ref):
    dxp, mn = xt_ref.shape
    mm, gi, gjdz = out_ref.shape
    n = mn // mm
    dz = zt_ref.shape[0]
    gj = gjdz // dz
    step0 = 1.0 / (gi - 1)
    step1 = 1.0 / (gj - 1)

    p = ls_ref[0, :]
    ls = 1e-5 + jnp.logaddexp(p, 0.0)
    inv = 1.0 / (ls * ls)
    inv0 = inv[0]
    inv1 = inv[1]

    x0_row = xt_ref[0:1, :]
    x1_col = jnp.transpose(xt_ref[1:2, :], (1, 0))

    jlane = jax.lax.broadcasted_iota(jnp.int32, (1, gi), 1)
    ax1_row = jlane.astype(jnp.float32) * step1
    ax0_col = jax.lax.broadcasted_iota(
        jnp.int32, (gi, 1), 0).astype(jnp.float32) * step0

    d0 = ax0_col - x0_row
    a = jnp.exp(-0.5 * inv0 * d0 * d0)

    d1 = x1_col - ax1_row
    bs = jnp.exp(-0.5 * inv1 * d1 * d1)

    q2 = jax.lax.broadcasted_iota(jnp.int32, (gj, gjdz), 1)
    r2 = jax.lax.broadcasted_iota(jnp.int32, (gj, gjdz), 0)
    e2 = jnp.where((q2 // dz) == r2, 1.0, 0.0)
    q3 = jax.lax.broadcasted_iota(jnp.int32, (dz, 128), 1)
    r3 = jax.lax.broadcasted_iota(jnp.int32, (dz, 128), 0)
    e3 = jnp.where((q3 % dz) == r3, 1.0, 0.0)

    b_rep = jnp.dot(bs, e2, preferred_element_type=jnp.float32,
                    precision=jax.lax.Precision.DEFAULT)
    z128 = jax.lax.dot_general(
        zt_ref[...], e3, (((0,), (0,)), ((), ())),
        preferred_element_type=jnp.float32,
        precision=jax.lax.Precision.DEFAULT)
    z_tile = jnp.concatenate([z128] * (gjdz // 128), axis=1)

    t = b_rep * z_tile

    qg = jax.lax.broadcasted_iota(jnp.int32, (gi, gj * dxp), 1)
    rg = jax.lax.broadcasted_iota(jnp.int32, (gi, gj * dxp), 0)
    gx = jnp.where(qg % dxp == 0, rg.astype(jnp.float32) * step0,
                   (qg // dxp).astype(jnp.float32) * step1)

    for b in range(mm):
        out_ref[b] = jax.lax.dot_general(
            a[:, b * n:(b + 1) * n], t[b * n:(b + 1) * n],
            (((1,), (0,)), ((), ())),
            preferred_element_type=jnp.float32,
            precision=jax.lax.Precision.DEFAULT)
        xg_ref[b] = gx


@functools.partial(jax.jit, static_argnames=())
def kernel(x, z, lengthscale_param):
    m, n, dx = x.shape
    dz = z.shape[-1]
    gi, gj = _GRID_SHAPE

    xt = x.reshape(m * n, dx).T
    zt = z.reshape(m * n, dz).T
    ls2 = lengthscale_param.reshape(1, dx)

    xg, out = pl.pallas_call(
        _setconv_kernel,
        in_specs=[
            pl.BlockSpec((dx, m * n), lambda: (0, 0)),
            pl.BlockSpec((dz, m * n), lambda: (0, 0)),
            pl.BlockSpec((1, dx), lambda: (0, 0)),
        ],
        out_specs=[
            pl.BlockSpec((m, gi, gj * dx), lambda: (0, 0, 0)),
            pl.BlockSpec((m, gi, gj * dz), lambda: (0, 0, 0)),
        ],
        out_shape=[
            jax.ShapeDtypeStruct((m, gi, gj * dx), jnp.float32),
            jax.ShapeDtypeStruct((m, gi, gj * dz), jnp.float32),
        ],
    )(xt, zt, ls2)

    x_grid = xg.reshape(m, gi, gj, dx)
    z_grid = out.reshape(m, gi, gj, dz)
    return (x_grid, z_grid)
